# Initial kernel scaffold; baseline (speedup 1.0000x reference)
import jax
import jax.numpy as jnp
from jax.experimental import pallas as pl


def kernel(x, edge_index, W1, b1, W2, b2):
    raise NotImplementedError("write your pallas kernel here")


# speedup vs baseline: 13.4399x; 13.4399x over previous
import functools

import jax
import jax.numpy as jnp
from jax import lax
from jax.experimental import pallas as pl
from jax.experimental.pallas import tpu as pltpu
from jax.experimental.pallas import tpu_sc as plsc

N_NODES = 10000
D = 128
N_EDGES = 320000

NPAD = 10240
NW = 32
CH = 128
NCH = 79
EPT = NCH * CH
EPAD = NW * EPT
TILES = 16
RPT = NPAD // TILES
DEGW = 16

_mesh = plsc.VectorSubcoreMesh(core_axis_name="c", subcore_axis_name="s")


def _zero_vmem_f32(ref, nrow, ncol):
    zero16 = jnp.zeros((16,), jnp.float32)

    def body(i, carry):
        r = i // (ncol // 16)
        k = i % (ncol // 16)
        ref[r, pl.ds(k * 16, 16)] = zero16
        return carry

    lax.fori_loop(0, nrow * (ncol // 16), body, 0)


@functools.partial(
    pl.kernel,
    mesh=_mesh,
    out_type=jax.ShapeDtypeStruct((2, NPAD, DEGW), jnp.float32),
    scratch_types=[
        pltpu.VMEM((NCH, CH), jnp.int32),
        pltpu.VMEM((CH, DEGW), jnp.float32),
        pltpu.VMEM((CH, DEGW), jnp.float32),
        pltpu.VMEM_SHARED((NPAD, DEGW), jnp.float32),
        pltpu.SemaphoreType.DMA,
    ],
)
def _deg_call(dst_hbm, out_hbm, dst_v, ones_v, zbuf_v, deg_sh, sem):
    c = lax.axis_index("c")
    s = lax.axis_index("s")
    wid = c * TILES + s

    idx16 = lax.iota(jnp.int32, 16)
    onesrow = jnp.where(idx16 == 0, jnp.float32(1.0), jnp.float32(0.0))

    def fill(i, carry):
        ones_v[i, pl.ds(0, 16)] = onesrow
        return carry

    lax.fori_loop(0, CH, fill, 0)
    _zero_vmem_f32(zbuf_v, CH, DEGW)

    row0 = s * RPT
    for j in range(RPT // CH):
        pltpu.sync_copy(zbuf_v, deg_sh.at[pl.ds(row0 + j * CH, CH)])
    plsc.subcore_barrier()

    pltpu.sync_copy(dst_hbm.at[wid], dst_v)

    def body(ci, carry):
        pltpu.sync_copy(ones_v, deg_sh.at[dst_v.at[ci]], add=True)
        return carry

    lax.fori_loop(0, NCH, body, 0)
    plsc.subcore_barrier()

    for j in range(RPT // CH):
        r0 = row0 + j * CH
        pltpu.sync_copy(deg_sh.at[pl.ds(r0, CH)], zbuf_v)
        pltpu.sync_copy(zbuf_v, out_hbm.at[c, pl.ds(r0, CH)])


@functools.partial(
    pl.kernel,
    mesh=_mesh,
    out_type=jax.ShapeDtypeStruct((2, NPAD, D), jnp.float32),
    scratch_types=[
        pltpu.VMEM((NCH, CH), jnp.int32),
        pltpu.VMEM((NCH, CH), jnp.int32),
        pltpu.VMEM((CH, D), jnp.float32),
        pltpu.VMEM_SHARED((NPAD, D), jnp.float32),
        pltpu.SemaphoreType.DMA,
    ],
)
def _msg_call(y_hbm, src_hbm, dst_hbm, out_hbm,
              src_v, dst_v, rows_v, acc_sh, sem0):
    c = lax.axis_index("c")
    s = lax.axis_index("s")
    wid = c * TILES + s

    _zero_vmem_f32(rows_v, CH, D)
    row0 = s * RPT
    for j in range(RPT // CH):
        pltpu.sync_copy(rows_v, acc_sh.at[pl.ds(row0 + j * CH, CH)])
    plsc.subcore_barrier()

    pltpu.sync_copy(src_hbm.at[wid], src_v)
    pltpu.sync_copy(dst_hbm.at[wid], dst_v)

    def body(ci, carry):
        pltpu.async_copy(y_hbm.at[src_v.at[ci]], rows_v, sem0).wait()
        pltpu.sync_copy(rows_v, acc_sh.at[dst_v.at[ci]], add=True)
        return carry

    lax.fori_loop(0, NCH, body, 0)
    plsc.subcore_barrier()

    for j in range(RPT // CH):
        r0 = row0 + j * CH
        pltpu.sync_copy(acc_sh.at[pl.ds(r0, CH)], rows_v)
        pltpu.sync_copy(rows_v, out_hbm.at[c, pl.ds(r0, CH)])


BR = 1280


def _tc1_body(x_ref, w_ref, cnt_ref, y_ref, d_ref):
    cnt = cnt_ref[0, :, 0:1] + cnt_ref[1, :, 0:1]
    d = lax.rsqrt(cnt + 1.0)
    y_ref[...] = jnp.dot(x_ref[...], w_ref[...],
                         preferred_element_type=jnp.float32) * d
    d_ref[...] = jnp.broadcast_to(d, (BR, DEGW))


def _tc2_body(acc_ref, y1_ref, d_ref, b1_ref, w2_ref, y2_ref):
    d = d_ref[:, 0:1]
    h = jnp.maximum(
        (acc_ref[0] + acc_ref[1] + y1_ref[...]) * d + b1_ref[...], 0.0)
    y2_ref[...] = jnp.dot(h, w2_ref[...],
                          preferred_element_type=jnp.float32) * d


def _tc3_body(acc_ref, y2_ref, d_ref, b2_ref, o_ref):
    d = d_ref[:, 0:1]
    o_ref[...] = (acc_ref[0] + acc_ref[1] + y2_ref[...]) * d + b2_ref[...]


_tc1 = pl.pallas_call(
    _tc1_body,
    grid=(NPAD // BR,),
    in_specs=[
        pl.BlockSpec((BR, D), lambda i: (i, 0)),
        pl.BlockSpec((D, D), lambda i: (0, 0)),
        pl.BlockSpec((2, BR, DEGW), lambda i: (0, i, 0)),
    ],
    out_specs=[
        pl.BlockSpec((BR, D), lambda i: (i, 0)),
        pl.BlockSpec((BR, DEGW), lambda i: (i, 0)),
    ],
    out_shape=[
        jax.ShapeDtypeStruct((NPAD, D), jnp.float32),
        jax.ShapeDtypeStruct((NPAD, DEGW), jnp.float32),
    ],
)

_tc2 = pl.pallas_call(
    _tc2_body,
    grid=(NPAD // BR,),
    in_specs=[
        pl.BlockSpec((2, BR, D), lambda i: (0, i, 0)),
        pl.BlockSpec((BR, D), lambda i: (i, 0)),
        pl.BlockSpec((BR, DEGW), lambda i: (i, 0)),
        pl.BlockSpec((1, D), lambda i: (0, 0)),
        pl.BlockSpec((D, D), lambda i: (0, 0)),
    ],
    out_specs=pl.BlockSpec((BR, D), lambda i: (i, 0)),
    out_shape=jax.ShapeDtypeStruct((NPAD, D), jnp.float32),
)

_tc3 = pl.pallas_call(
    _tc3_body,
    grid=(NPAD // BR,),
    in_specs=[
        pl.BlockSpec((2, BR, D), lambda i: (0, i, 0)),
        pl.BlockSpec((BR, D), lambda i: (i, 0)),
        pl.BlockSpec((BR, DEGW), lambda i: (i, 0)),
        pl.BlockSpec((1, D), lambda i: (0, 0)),
    ],
    out_specs=pl.BlockSpec((BR, D), lambda i: (i, 0)),
    out_shape=jax.ShapeDtypeStruct((NPAD, D), jnp.float32),
)


@jax.jit
def kernel(x, edge_index, W1, b1, W2, b2):
    src = edge_index[0].astype(jnp.int32)
    dst = edge_index[1].astype(jnp.int32)
    pad = jnp.full((EPAD - N_EDGES,), N_NODES, jnp.int32)
    src3 = jnp.concatenate([src, pad]).reshape(NW, NCH, CH)
    dst3 = jnp.concatenate([dst, pad]).reshape(NW, NCH, CH)

    x_pad = jnp.zeros((NPAD, D), jnp.float32).at[:N_NODES].set(x)

    cnt = _deg_call(dst3)
    y1, dmat = _tc1(x_pad, W1, cnt)
    acc1 = _msg_call(y1, src3, dst3)
    y2 = _tc2(acc1, y1, dmat, b1.reshape(1, D), W2)
    acc2 = _msg_call(y2, src3, dst3)
    out = _tc3(acc2, y2, dmat, b2.reshape(1, D))
    return out[:N_NODES]
